# Initial kernel scaffold; baseline (speedup 1.0000x reference)
import jax
import jax.numpy as jnp
from jax.experimental import pallas as pl


def kernel(edges, features, label_masks, batch, W1, b1, W2, b2, Wp, bp):
    raise NotImplementedError("write your pallas kernel here")


# speedup vs baseline: 5.4379x; 5.4379x over previous
import functools

import jax
import jax.numpy as jnp
from jax import lax
from jax.experimental import pallas as pl
from jax.experimental.pallas import tpu as pltpu
from jax.experimental.pallas import tpu_sc as plsc

N = 10000
E = 320000
D = 128
G = 128
OUTC = 1

NC = 2
NS = 16
NW = NC * NS

CHUNK = 128
NCHUNK = 80
EPW = NCHUNK * CHUNK
EPAD = NW * EPW
NBUF = 2

N_ACC = 10240
ZROWS = N_ACC // NS

NPAD = 10240
NPT = NPAD // NW
G_PAD = 160

_mesh = plsc.VectorSubcoreMesh(
    core_axis_name="c", subcore_axis_name="s", num_cores=NC, num_subcores=NS)


def _h0_body(f_ref, m_ref, w_ref, o_ref):
    x = f_ref[...] * m_ref[...]
    o_ref[...] = jnp.dot(x, w_ref[...], preferred_element_type=jnp.float32)


_h0_call = pl.pallas_call(
    _h0_body, out_shape=jax.ShapeDtypeStruct((N, D), jnp.float32))


def _edge_agg_body(h_hbm, src_hbm, dst_hbm, z_hbm, out_hbm,
                   acc, src_v, dring, rowbuf, gsem, ssem, isem):
    c = lax.axis_index("c")
    s = lax.axis_index("s")
    wid = c * NS + s

    pltpu.sync_copy(src_hbm.at[wid], src_v)
    pltpu.sync_copy(z_hbm, acc.at[pl.ds(s * ZROWS, ZROWS)])
    plsc.subcore_barrier()

    def start_gather(j, b):
        pltpu.async_copy(h_hbm.at[src_v.at[pl.ds(j * CHUNK, CHUNK)]],
                         rowbuf.at[b], gsem.at[b])

    def wait_gather(j, b):
        pltpu.make_async_copy(h_hbm.at[src_v.at[pl.ds(j * CHUNK, CHUNK)]],
                              rowbuf.at[b], gsem.at[b]).wait()

    def start_idx(j, b):
        pltpu.async_copy(dst_hbm.at[wid, j], dring.at[b], isem.at[b])

    def wait_idx(j, b):
        pltpu.make_async_copy(dst_hbm.at[wid, j], dring.at[b],
                              isem.at[b]).wait()

    def start_scatter(b):
        pltpu.async_copy(rowbuf.at[b], acc.at[dring.at[b]], ssem.at[b],
                         add=True)

    def wait_scatter(b):
        pltpu.make_async_copy(rowbuf.at[b], acc.at[dring.at[b]],
                              ssem.at[b]).wait()

    for b in range(NBUF):
        start_gather(b, b)
        start_idx(b, b)

    def ring(g, carry):
        for b in range(NBUF):
            j = g * NBUF + b
            wait_gather(j, b)
            wait_idx(j, b)
            start_scatter(b)
            wait_scatter(b)
            start_gather(j + NBUF, b)
            start_idx(j + NBUF, b)
        return carry

    lax.fori_loop(0, NCHUNK // NBUF - 1, ring, 0)
    for b in range(NBUF):
        j = NCHUNK - NBUF + b
        wait_gather(j, b)
        wait_idx(j, b)
        start_scatter(b)
        wait_scatter(b)

    plsc.subcore_barrier()
    wbase = s * ZROWS
    pltpu.sync_copy(acc.at[pl.ds(wbase, ZROWS)],
                    out_hbm.at[c, pl.ds(wbase, ZROWS)])


_edge_agg_call = pl.kernel(
    _edge_agg_body,
    out_type=jax.ShapeDtypeStruct((NC, N_ACC, D), jnp.float32),
    mesh=_mesh,
    scratch_types=[
        pltpu.VMEM_SHARED((N_ACC, D), jnp.float32),
        pltpu.VMEM((EPW,), jnp.int32),
        pltpu.VMEM((NBUF, CHUNK), jnp.int32),
        pltpu.VMEM((NBUF, CHUNK, D), jnp.float32),
        pltpu.SemaphoreType.DMA((NBUF,)),
        pltpu.SemaphoreType.DMA((NBUF,)),
        pltpu.SemaphoreType.DMA((NBUF,)),
    ],
)


def _y_body(h_ref, a_ref, b1_ref, w2_ref, wp_ref, y_ref):
    x1 = jnp.maximum(
        h_ref[...] + a_ref[0, :N, :] + a_ref[1, :N, :] + b1_ref[...], 0.0)
    w2p = jnp.dot(w2_ref[...], wp_ref[...], preferred_element_type=jnp.float32)
    y_ref[...] = jnp.dot(x1, w2p, preferred_element_type=jnp.float32)


_y_call = pl.pallas_call(
    _y_body, out_shape=jax.ShapeDtypeStruct((N, OUTC), jnp.float32))


def _scalar_body(y_hbm, bat_hbm, srcf_hbm, dstf_hbm, z_hbm, out_hbm,
                 y_v, bat_v, src_v, dst_v, accg, cntg, outb):
    c = lax.axis_index("c")
    s = lax.axis_index("s")
    wid = c * NS + s

    pltpu.sync_copy(y_hbm, y_v)
    pltpu.sync_copy(bat_hbm, bat_v)
    pltpu.sync_copy(srcf_hbm.at[wid], src_v)
    pltpu.sync_copy(dstf_hbm.at[wid], dst_v)
    pltpu.sync_copy(z_hbm, accg)
    pltpu.sync_copy(z_hbm, cntg)

    lanes = lax.iota(jnp.int32, 16)
    ones = jnp.ones((16,), jnp.float32)

    def eloop(i, carry):
        si = src_v[pl.ds(i * 16, 16)]
        di = dst_v[pl.ds(i * 16, 16)]
        vals = plsc.load_gather(y_v, [si])
        gs = plsc.load_gather(bat_v, [di])
        plsc.addupdate_scatter(accg, [lanes, gs], vals)
        return carry

    lax.fori_loop(0, EPW // 16, eloop, 0)

    nbase = wid * NPT

    def nloop(i, carry):
        yv = y_v[pl.ds(nbase + i * 16, 16)]
        gv = bat_v[pl.ds(nbase + i * 16, 16)]
        plsc.addupdate_scatter(accg, [lanes, gv], yv)
        plsc.addupdate_scatter(cntg, [lanes, gv], ones)
        return carry

    lax.fori_loop(0, NPT // 16, nloop, 0)

    for cc in range(G_PAD // 16):
        sa = accg[0, pl.ds(cc * 16, 16)]
        sn = cntg[0, pl.ds(cc * 16, 16)]
        for r in range(1, 16):
            sa = sa + accg[r, pl.ds(cc * 16, 16)]
            sn = sn + cntg[r, pl.ds(cc * 16, 16)]
        outb[0, pl.ds(cc * 16, 16)] = sa
        outb[1, pl.ds(cc * 16, 16)] = sn

    pltpu.sync_copy(outb, out_hbm.at[c, s])


_scalar_call = pl.kernel(
    _scalar_body,
    out_type=jax.ShapeDtypeStruct((NC, NS, 2, G_PAD), jnp.float32),
    mesh=_mesh,
    compiler_params=pltpu.CompilerParams(needs_layout_passes=False),
    scratch_types=[
        pltpu.VMEM((NPAD,), jnp.float32),
        pltpu.VMEM((NPAD,), jnp.int32),
        pltpu.VMEM((EPW,), jnp.int32),
        pltpu.VMEM((EPW,), jnp.int32),
        pltpu.VMEM((16, G_PAD), jnp.float32),
        pltpu.VMEM((16, G_PAD), jnp.float32),
        pltpu.VMEM((2, G_PAD), jnp.float32),
    ],
)


def _final_body(p_ref, b2_ref, wp_ref, bp_ref, o_ref):
    esum = jnp.sum(p_ref[0], axis=0, keepdims=True)
    csum = jnp.sum(p_ref[1], axis=0, keepdims=True)
    c2 = jnp.dot(b2_ref[...], wp_ref[...], preferred_element_type=jnp.float32)
    e = esum[:, :G]
    n = csum[:, :G]
    o_ref[...] = (e + n * c2) / jnp.maximum(n, 1.0) + bp_ref[...]


_final_call = pl.pallas_call(
    _final_body, out_shape=jax.ShapeDtypeStruct((1, G), jnp.float32))


def kernel(edges, features, label_masks, batch, W1, b1, W2, b2, Wp, bp):
    src = edges[0]
    dst = edges[1]
    pad = EPAD - E
    src_p = jnp.concatenate([src, jnp.zeros((pad,), jnp.int32)])
    dst_p = jnp.concatenate([dst, jnp.full((pad,), N, jnp.int32)])
    src_t = src_p.reshape(NW, NCHUNK, CHUNK)
    dst_t = dst_p.reshape(NW, NCHUNK, CHUNK)
    src_f = src_p.reshape(NW, EPW)
    dst_f = dst_p.reshape(NW, EPW)
    bat_p = jnp.concatenate([batch, jnp.full((NPAD - N,), G, jnp.int32)])

    zrows = jnp.zeros((ZROWS, D), jnp.float32)
    zsmall = jnp.zeros((16, G_PAD), jnp.float32)

    h0 = _h0_call(features, label_masks, W1)
    agg = _edge_agg_call(h0, src_f, dst_t, zrows)
    y = _y_call(h0, agg, b1.reshape(1, D), W2, Wp)
    y_pad = jnp.concatenate([y[:, 0], jnp.zeros((NPAD - N,), jnp.float32)])
    p = _scalar_call(y_pad, bat_p, src_f, dst_f, zsmall)
    p2 = p.transpose(2, 0, 1, 3).reshape(2, NW, G_PAD)
    out = _final_call(p2, b2.reshape(1, D), Wp, bp.reshape(1, 1))
    return out.reshape(G, OUTC)
